# Initial kernel scaffold; baseline (speedup 1.0000x reference)
import jax
import jax.numpy as jnp
from jax.experimental import pallas as pl


def kernel(x, pos, batch, ptr, params):
    raise NotImplementedError("write your pallas kernel here")


# speedup vs baseline: 1.5075x; 1.5075x over previous
import functools

import jax
import jax.numpy as jnp
from jax.experimental import pallas as pl

K_NBR = 16
N_PTS = 8192


def _fc0_body(x_ref, w_ref, b_ref, o_ref):
    o_ref[...] = jnp.dot(x_ref[...], w_ref[...],
                         preferred_element_type=jnp.float32) + b_ref[...]


def _fc0(x, w, b):
    n, di = x.shape
    do = w.shape[1]
    return pl.pallas_call(
        _fc0_body,
        out_shape=jax.ShapeDtypeStruct((n, do), jnp.float32),
    )(x, w, b.reshape(1, do))


def _knn(query, ref, k):
    ref_sq = jnp.sum(ref * ref, axis=1)
    d = (jnp.sum(query * query, axis=1, keepdims=True)
         - 2.0 * (query @ ref.T) + ref_sq[None, :])
    return jax.lax.top_k(-d, k)[1]


def _lin(pp, x):
    y = x @ pp["W"]
    if "b" in pp:
        y = y + pp["b"]
    return y


def _lfa_dense(pp, x, pos, nbr):
    n, k = nbr.shape
    pos_j = pos[nbr]
    pos_i = pos[:, None, :]
    d = pos_j - pos_i
    eu = jnp.sqrt(jnp.sum(d * d, axis=2, keepdims=True) + 1e-12)
    rel = jnp.concatenate(
        [jnp.broadcast_to(pos_i, (n, k, 3)), pos_j, d, eu], axis=2)
    enc = jax.nn.relu(rel @ pp["enc"]["W"] + pp["enc"]["b"])
    local = jnp.concatenate([x[nbr], enc], axis=2)
    att = local @ pp["att"]["W"]
    att = att - jnp.max(att, axis=1, keepdims=True)
    e = jnp.exp(att)
    sm = e / jnp.sum(e, axis=1, keepdims=True)
    agg = jnp.sum(sm * local, axis=1)
    return jax.nn.relu(_lin(pp["post"], agg))


def _drb(pp, x, pos):
    nbr = _knn(pos, pos, K_NBR)
    sc = _lin(pp["short"], x)
    h = jax.nn.relu(_lin(pp["mlp1"], x))
    h = _lfa_dense(pp["lfa1"], h, pos, nbr)
    h = _lfa_dense(pp["lfa2"], h, pos, nbr)
    h = _lin(pp["mlp2"], h)
    return jax.nn.leaky_relu(h + sc, 0.01)


def _decim(n, salt):
    perm = jax.random.permutation(jax.random.fold_in(jax.random.key(7), salt), n)
    return perm[: n // 4]


def _fp(pp, x_in, pos_in, x_skip, pos_skip):
    ni = _knn(pos_skip, pos_in, 1)[:, 0]
    return jax.nn.relu(_lin(pp, jnp.concatenate([x_in[ni], x_skip], axis=1)))


def kernel(x, pos, batch, ptr, params):
    n = x.shape[0]
    h = _fc0(x, params["fc0"]["W"], params["fc0"]["b"])
    h1 = _drb(params["b1"], h, pos)
    i1 = _decim(n, 1)
    h1d = h1[i1]
    p1d = pos[i1]
    h2 = _drb(params["b2"], h1d, p1d)
    i2 = _decim(n // 4, 2)
    h2d = h2[i2]
    p2d = p1d[i2]
    h3 = _drb(params["b3"], h2d, p2d)
    i3 = _decim(n // 16, 3)
    h3d = h3[i3]
    p3d = p2d[i3]
    h4 = _drb(params["b4"], h3d, p3d)
    i4 = _decim(n // 64, 4)
    h4d = h4[i4]
    p4d = p3d[i4]
    seed_idx = jnp.arange(n)[i1][i2][i3][i4]
    s = jax.nn.relu(_lin(params["summit"], h4d))
    sem = _fp(params["sem_fp4"], s, p4d, h3d, p3d)
    sem = _fp(params["sem_fp3"], sem, p3d, h2d, p2d)
    sem = _fp(params["sem_fp2"], sem, p2d, h1d, p1d)
    sem = _fp(params["sem_fp1"], sem, p1d, h1, pos)
    semx = jax.nn.relu(_lin(params["clf1"], sem))
    semx = jax.nn.relu(_lin(params["clf2"], semx))
    sem_out = jax.nn.log_softmax(_lin(params["fc_classif"], semx), axis=-1)
    inst = _fp(params["inst_fp4"], s, p4d, h3d, p3d)
    inst = _fp(params["inst_fp3"], inst, p3d, h2d, p2d)
    inst = _fp(params["inst_fp2"], inst, p2d, h1d, p1d)
    inst = _fp(params["inst_fp1"], inst, p1d, h1, pos)
    inst_out = _lin(params["fc_inst"], inst)
    return sem_out, inst_out, seed_idx


# speedup vs baseline: 4.1347x; 2.7427x over previous
import functools

import jax
import jax.numpy as jnp
from jax import lax
from jax.experimental import pallas as pl
from jax.experimental.pallas import tpu as pltpu
from jax.experimental.pallas import tpu_sc as plsc

K_NBR = 16

_SC_NC = 2
_SC_NS = 16
_SC_NW = _SC_NC * _SC_NS


def _sc_gather(table, idx):
    (b0,) = idx.shape
    b_total = (b0 + 8 * _SC_NW - 1) // (8 * _SC_NW) * (8 * _SC_NW)
    if b_total != b0:
        idx = jnp.concatenate(
            [idx, jnp.zeros((b_total - b0,), idx.dtype)])
    n, d0 = table.shape
    d = (d0 + 127) // 128 * 128
    if d != d0:
        table = jnp.concatenate(
            [table, jnp.zeros((n, d - d0), jnp.float32)], axis=1)
    bpw = b_total // _SC_NW
    cs = min(bpw, max(8, 262144 // (d * 4)))
    while bpw % cs:
        cs -= 1
    iters = bpw // cs
    mesh = plsc.VectorSubcoreMesh(core_axis_name="c", subcore_axis_name="s")

    @functools.partial(
        pl.kernel,
        mesh=mesh,
        out_type=jax.ShapeDtypeStruct((b_total, d), jnp.float32),
        scratch_types=[
            pltpu.VMEM((cs,), jnp.int32),
            pltpu.VMEM((cs, d), jnp.float32),
            pltpu.SemaphoreType.DMA,
        ],
    )
    def k(table_hbm, idx_hbm, out_hbm, idx_v, rows_v, sem):
        wid = lax.axis_index("s") * _SC_NC + lax.axis_index("c")
        base = wid * bpw

        def body(t, carry):
            off = base + t * cs
            pltpu.sync_copy(idx_hbm.at[pl.ds(off, cs)], idx_v)
            pltpu.async_copy(table_hbm.at[idx_v], rows_v, sem).wait()
            pltpu.sync_copy(rows_v, out_hbm.at[pl.ds(off, cs)])
            return carry

        lax.fori_loop(0, iters, body, 0)

    out = k(table, idx)
    return out[:b0, :d0]


def _extract_min_cols(d, idx_of, k):
    big = jnp.int32(2**30)
    vcols, icols = [], []
    for j in range(k):
        m = jnp.min(d, axis=1, keepdims=True)
        sel = jnp.min(jnp.where(d <= m, idx_of, big), axis=1, keepdims=True)
        vcols.append(m)
        icols.append(sel)
        if j + 1 < k:
            d = jnp.where(idx_of == sel, jnp.float32(jnp.inf), d)
    if k == 1:
        return vcols[0], icols[0]
    return jnp.concatenate(vcols, axis=1), jnp.concatenate(icols, axis=1)


def _knn_body(k, chunk, q_ref, r_ref, o_ref):
    nr = r_ref.shape[0]
    q = q_ref[...]
    qsq = jnp.sum(q * q, axis=1, keepdims=True)
    cand_v, cand_i = [], []
    for c in range(nr // chunk):
        r = r_ref[pl.ds(c * chunk, chunk), :]
        rsq = jnp.sum(r * r, axis=1)[None, :]
        d = qsq - 2.0 * jnp.dot(q, r.T,
                                preferred_element_type=jnp.float32) + rsq
        iota = lax.broadcasted_iota(jnp.int32, d.shape, 1) + c * chunk
        kk = min(k, chunk)
        v, i = _extract_min_cols(d, iota, kk)
        cand_v.append(v)
        cand_i.append(i)
    if len(cand_v) == 1:
        o_ref[...] = cand_i[0][:, :k]
        return
    vs = jnp.concatenate(cand_v, axis=1)
    xs = jnp.concatenate(cand_i, axis=1)
    _, sel = _extract_min_cols(vs, xs, k)
    o_ref[...] = sel


def _knn(query, ref, k):
    nq = query.shape[0]
    nr = ref.shape[0]
    bm = min(nq, max(128, 2**22 // (4 * nr)))
    grid = nq // bm
    chunk = 512 if (nr % 512 == 0 and nr >= 512) else nr
    return pl.pallas_call(
        functools.partial(_knn_body, k, chunk),
        grid=(grid,),
        in_specs=[
            pl.BlockSpec((bm, 3), lambda i: (i, 0)),
            pl.BlockSpec((nr, 3), lambda i: (0, 0)),
        ],
        out_specs=pl.BlockSpec((bm, k), lambda i: (i, 0)),
        out_shape=jax.ShapeDtypeStruct((nq, k), jnp.int32),
    )(query, ref)


def _lfa_body(bm, cin, x_ref, xj_ref, pi_ref, pj_ref,
              wpi_ref, wpj_ref, wd_ref, weu_ref, eb_ref,
              axa_ref, axb_ref, aea_ref, aeb_ref,
              pwx_ref, pwe_ref, pb_ref, o_ref):
    k = K_NBR
    ch = cin
    c = 2 * cin
    pi = pi_ref[...]
    pj = pj_ref[...]
    dv = pj - pi[:, None, :]
    eu = jnp.sqrt(jnp.sum(dv * dv, axis=2, keepdims=True) + 1e-12)
    pj2 = pj.reshape(bm * k, 3)
    dv2 = dv.reshape(bm * k, 3)
    eu2 = eu.reshape(bm * k, 1)
    enc2 = (jnp.dot(pj2, wpj_ref[...], preferred_element_type=jnp.float32)
            + jnp.dot(dv2, wd_ref[...], preferred_element_type=jnp.float32)
            + eu2 * weu_ref[...]
            + eb_ref[...])
    enc_pi = jnp.dot(pi, wpi_ref[...], preferred_element_type=jnp.float32)
    enc = jnp.maximum(enc2.reshape(bm, k, ch) + enc_pi[:, None, :], 0.0)
    enc2 = enc.reshape(bm * k, ch)
    xj = xj_ref[...]
    xj2 = xj.reshape(bm * k, cin)

    att_a = (jnp.dot(xj2, axa_ref[...], preferred_element_type=jnp.float32)
             + jnp.dot(enc2, aea_ref[...], preferred_element_type=jnp.float32)
             ).reshape(bm, k, cin)
    att_b = (jnp.dot(xj2, axb_ref[...], preferred_element_type=jnp.float32)
             + jnp.dot(enc2, aeb_ref[...], preferred_element_type=jnp.float32)
             ).reshape(bm, k, ch)
    ea = jnp.exp(att_a - jnp.max(att_a, axis=1, keepdims=True))
    sa = ea / jnp.sum(ea, axis=1, keepdims=True)
    eb = jnp.exp(att_b - jnp.max(att_b, axis=1, keepdims=True))
    sb = eb / jnp.sum(eb, axis=1, keepdims=True)
    agg_x = jnp.sum(sa * xj, axis=1)
    agg_e = jnp.sum(sb * enc, axis=1)
    out = (jnp.dot(agg_x, pwx_ref[...], preferred_element_type=jnp.float32)
           + jnp.dot(agg_e, pwe_ref[...], preferred_element_type=jnp.float32)
           + pb_ref[...])
    o_ref[...] = jnp.maximum(out, 0.0)


def _lfa(pp, x, pos, flat_nbr):
    n, cin = x.shape
    c = 2 * cin
    table = jnp.concatenate([x, pos], axis=1)
    g = _sc_gather(table, flat_nbr)
    xj = g[:, :cin].reshape(n, K_NBR, cin)
    pj = g[:, cin:cin + 3].reshape(n, K_NBR, 3)

    ew, ebias = pp["enc"]["W"], pp["enc"]["b"].reshape(1, -1)
    aw = pp["att"]["W"]
    pw, pbias = pp["post"]["W"], pp["post"]["b"].reshape(1, -1)
    bm = min(n, 256)
    grid = n // bm
    row = lambda i: (i, 0)
    row3 = lambda i: (i, 0, 0)
    full = lambda i: (0, 0)
    return pl.pallas_call(
        functools.partial(_lfa_body, bm, cin),
        grid=(grid,),
        in_specs=[
            pl.BlockSpec((bm, cin), row),
            pl.BlockSpec((bm, K_NBR, cin), row3),
            pl.BlockSpec((bm, 3), row),
            pl.BlockSpec((bm, K_NBR, 3), row3),
            pl.BlockSpec((3, cin), full),
            pl.BlockSpec((3, cin), full),
            pl.BlockSpec((3, cin), full),
            pl.BlockSpec((1, cin), full),
            pl.BlockSpec((1, cin), full),
            pl.BlockSpec((cin, cin), full),
            pl.BlockSpec((cin, cin), full),
            pl.BlockSpec((cin, cin), full),
            pl.BlockSpec((cin, cin), full),
            pl.BlockSpec((cin, c), full),
            pl.BlockSpec((cin, c), full),
            pl.BlockSpec((1, c), full),
        ],
        out_specs=pl.BlockSpec((bm, c), row),
        out_shape=jax.ShapeDtypeStruct((n, c), jnp.float32),
    )(x, xj, pos, pj,
      ew[0:3], ew[3:6], ew[6:9], ew[9:10], ebias,
      aw[:cin, :cin], aw[:cin, cin:], aw[cin:, :cin], aw[cin:, cin:],
      pw[:cin], pw[cin:], pbias)


def _linear_body(act, x_ref, w_ref, b_ref, o_ref):
    y = jnp.dot(x_ref[...], w_ref[...],
                preferred_element_type=jnp.float32) + b_ref[...]
    if act == "relu":
        y = jnp.maximum(y, 0.0)
    elif act == "logsoftmax":
        y = y - jnp.max(y, axis=1, keepdims=True)
        y = y - jnp.log(jnp.sum(jnp.exp(y), axis=1, keepdims=True))
    o_ref[...] = y


def _linear(pp, x, act="none"):
    n = x.shape[0]
    do = pp["W"].shape[1]
    return pl.pallas_call(
        functools.partial(_linear_body, act),
        out_shape=jax.ShapeDtypeStruct((n, do), jnp.float32),
    )(x, pp["W"], pp["b"].reshape(1, do))


def _drb_tail_body(h_ref, x_ref, w2_ref, b2_ref, ws_ref, bs_ref, o_ref):
    y = (jnp.dot(h_ref[...], w2_ref[...], preferred_element_type=jnp.float32)
         + b2_ref[...]
         + jnp.dot(x_ref[...], ws_ref[...], preferred_element_type=jnp.float32)
         + bs_ref[...])
    o_ref[...] = jnp.where(y > 0, y, 0.01 * y)


def _drb_tail(pp2, pps, h, x):
    n = h.shape[0]
    do = pp2["W"].shape[1]
    return pl.pallas_call(
        _drb_tail_body,
        out_shape=jax.ShapeDtypeStruct((n, do), jnp.float32),
    )(h, x, pp2["W"], pp2["b"].reshape(1, do),
      pps["W"], pps["b"].reshape(1, do))


def _fp_body(xg_ref, xs_ref, w1_ref, w2_ref, b_ref, o_ref):
    y = (jnp.dot(xg_ref[...], w1_ref[...], preferred_element_type=jnp.float32)
         + jnp.dot(xs_ref[...], w2_ref[...], preferred_element_type=jnp.float32)
         + b_ref[...])
    o_ref[...] = jnp.maximum(y, 0.0)


def _fp(pp, xg, x_skip):
    n, din = xg.shape
    do = pp["W"].shape[1]
    return pl.pallas_call(
        _fp_body,
        out_shape=jax.ShapeDtypeStruct((n, do), jnp.float32),
    )(xg, x_skip, pp["W"][:din], pp["W"][din:], pp["b"].reshape(1, do))


def _drb(pp, x, pos):
    nbr = _knn(pos, pos, K_NBR)
    flat = nbr.reshape(-1)
    h = _linear(pp["mlp1"], x, act="relu")
    h = _lfa(pp["lfa1"], h, pos, flat)
    h = _lfa(pp["lfa2"], h, pos, flat)
    return _drb_tail(pp["mlp2"], pp["short"], h, x)


def _decim(n, salt):
    perm = jax.random.permutation(
        jax.random.fold_in(jax.random.key(7), salt), n)
    return perm[: n // 4]


def kernel(x, pos, batch, ptr, params):
    n = x.shape[0]
    h = _linear(params["fc0"], x)
    h1 = _drb(params["b1"], h, pos)
    i1 = _decim(n, 1)
    h1d = h1[i1]
    p1d = pos[i1]
    h2 = _drb(params["b2"], h1d, p1d)
    i2 = _decim(n // 4, 2)
    h2d = h2[i2]
    p2d = p1d[i2]
    h3 = _drb(params["b3"], h2d, p2d)
    i3 = _decim(n // 16, 3)
    h3d = h3[i3]
    p3d = p2d[i3]
    h4 = _drb(params["b4"], h3d, p3d)
    i4 = _decim(n // 64, 4)
    h4d = h4[i4]
    p4d = p3d[i4]
    seed_idx = jnp.arange(n)[i1][i2][i3][i4]
    s = _linear(params["summit"], h4d, act="relu")

    ni4 = _knn(p3d, p4d, 1)[:, 0]
    ni3 = _knn(p2d, p3d, 1)[:, 0]
    ni2 = _knn(p1d, p2d, 1)[:, 0]
    ni1 = _knn(pos, p1d, 1)[:, 0]

    sem = _fp(params["sem_fp4"], _sc_gather(s, ni4), h3d)
    sem = _fp(params["sem_fp3"], _sc_gather(sem, ni3), h2d)
    sem = _fp(params["sem_fp2"], _sc_gather(sem, ni2), h1d)
    sem = _fp(params["sem_fp1"], _sc_gather(sem, ni1), h1)
    semx = _linear(params["clf1"], sem, act="relu")
    semx = _linear(params["clf2"], semx, act="relu")
    sem_out = _linear(params["fc_classif"], semx, act="logsoftmax")

    inst = _fp(params["inst_fp4"], _sc_gather(s, ni4), h3d)
    inst = _fp(params["inst_fp3"], _sc_gather(inst, ni3), h2d)
    inst = _fp(params["inst_fp2"], _sc_gather(inst, ni2), h1d)
    inst = _fp(params["inst_fp1"], _sc_gather(inst, ni1), h1)
    inst_out = _linear(params["fc_inst"], inst)
    return sem_out, inst_out, seed_idx


# speedup vs baseline: 4.1408x; 1.0015x over previous
import functools

import jax
import jax.numpy as jnp
from jax import lax
from jax.experimental import pallas as pl
from jax.experimental.pallas import tpu as pltpu
from jax.experimental.pallas import tpu_sc as plsc

K_NBR = 16

_SC_NC = 2
_SC_NS = 16
_SC_NW = _SC_NC * _SC_NS


def _sc_gather(table, idx):
    (b0,) = idx.shape
    b_total = (b0 + 8 * _SC_NW - 1) // (8 * _SC_NW) * (8 * _SC_NW)
    if b_total != b0:
        idx = jnp.concatenate(
            [idx, jnp.zeros((b_total - b0,), idx.dtype)])
    n, d0 = table.shape
    d = (d0 + 127) // 128 * 128
    if d != d0:
        table = jnp.concatenate(
            [table, jnp.zeros((n, d - d0), jnp.float32)], axis=1)
    bpw = b_total // _SC_NW
    cs = min(bpw, max(8, 262144 // (d * 4)))
    while bpw % cs:
        cs -= 1
    iters = bpw // cs
    mesh = plsc.VectorSubcoreMesh(core_axis_name="c", subcore_axis_name="s")

    @functools.partial(
        pl.kernel,
        mesh=mesh,
        out_type=jax.ShapeDtypeStruct((b_total, d), jnp.float32),
        scratch_types=[
            pltpu.VMEM((cs,), jnp.int32),
            pltpu.VMEM((cs, d), jnp.float32),
            pltpu.SemaphoreType.DMA,
        ],
    )
    def k(table_hbm, idx_hbm, out_hbm, idx_v, rows_v, sem):
        wid = lax.axis_index("s") * _SC_NC + lax.axis_index("c")
        base = wid * bpw

        def body(t, carry):
            off = base + t * cs
            pltpu.sync_copy(idx_hbm.at[pl.ds(off, cs)], idx_v)
            pltpu.async_copy(table_hbm.at[idx_v], rows_v, sem).wait()
            pltpu.sync_copy(rows_v, out_hbm.at[pl.ds(off, cs)])
            return carry

        lax.fori_loop(0, iters, body, 0)

    out = k(table, idx)
    return out[:b0, :d0]


def _extract_min_cols(d, idx_of, k):
    big = jnp.int32(2**30)
    vcols, icols = [], []
    for j in range(k):
        m = jnp.min(d, axis=1, keepdims=True)
        sel = jnp.min(jnp.where(d <= m, idx_of, big), axis=1, keepdims=True)
        vcols.append(m)
        icols.append(sel)
        if j + 1 < k:
            d = jnp.where(idx_of == sel, jnp.float32(jnp.inf), d)
    if k == 1:
        return vcols[0], icols[0]
    return jnp.concatenate(vcols, axis=1), jnp.concatenate(icols, axis=1)


def _knn_body(k, chunk, q_ref, r_ref, o_ref):
    nr = r_ref.shape[0]
    q = q_ref[...]
    qsq = jnp.sum(q * q, axis=1, keepdims=True)
    cand_v, cand_i = [], []
    for c in range(nr // chunk):
        r = r_ref[pl.ds(c * chunk, chunk), :]
        rsq = jnp.sum(r * r, axis=1)[None, :]
        d = qsq - 2.0 * jnp.dot(q, r.T,
                                preferred_element_type=jnp.float32) + rsq
        iota = lax.broadcasted_iota(jnp.int32, d.shape, 1) + c * chunk
        kk = min(k, chunk)
        v, i = _extract_min_cols(d, iota, kk)
        cand_v.append(v)
        cand_i.append(i)
    if len(cand_v) == 1:
        o_ref[...] = cand_i[0][:, :k]
        return
    vs = jnp.concatenate(cand_v, axis=1)
    xs = jnp.concatenate(cand_i, axis=1)
    _, sel = _extract_min_cols(vs, xs, k)
    o_ref[...] = sel


def _knn(query, ref, k):
    nq = query.shape[0]
    nr = ref.shape[0]
    bm = min(nq, max(128, 2**22 // (4 * nr)))
    grid = nq // bm
    chunk = 512 if (nr % 512 == 0 and nr >= 512) else nr
    return pl.pallas_call(
        functools.partial(_knn_body, k, chunk),
        grid=(grid,),
        in_specs=[
            pl.BlockSpec((bm, 3), lambda i: (i, 0)),
            pl.BlockSpec((nr, 3), lambda i: (0, 0)),
        ],
        out_specs=pl.BlockSpec((bm, k), lambda i: (i, 0)),
        out_shape=jax.ShapeDtypeStruct((nq, k), jnp.int32),
    )(query, ref)


def _lfa_body(bm, cin, x_ref, xj_ref, pi_ref, pj_ref,
              wpi_ref, wpj_ref, wd_ref, weu_ref, eb_ref,
              axa_ref, axb_ref, aea_ref, aeb_ref,
              pwx_ref, pwe_ref, pb_ref, o_ref):
    k = K_NBR
    ch = cin
    c = 2 * cin
    pi = pi_ref[...]
    pj = pj_ref[...]
    dv = pj - pi[:, None, :]
    eu = jnp.sqrt(jnp.sum(dv * dv, axis=2, keepdims=True) + 1e-12)
    pj2 = pj.reshape(bm * k, 3)
    dv2 = dv.reshape(bm * k, 3)
    eu2 = eu.reshape(bm * k, 1)
    enc2 = (jnp.dot(pj2, wpj_ref[...], preferred_element_type=jnp.float32)
            + jnp.dot(dv2, wd_ref[...], preferred_element_type=jnp.float32)
            + eu2 * weu_ref[...]
            + eb_ref[...])
    enc_pi = jnp.dot(pi, wpi_ref[...], preferred_element_type=jnp.float32)
    enc = jnp.maximum(enc2.reshape(bm, k, ch) + enc_pi[:, None, :], 0.0)
    enc2 = enc.reshape(bm * k, ch)
    xj = xj_ref[...]
    xj2 = xj.reshape(bm * k, cin)

    att_a = (jnp.dot(xj2, axa_ref[...], preferred_element_type=jnp.float32)
             + jnp.dot(enc2, aea_ref[...], preferred_element_type=jnp.float32)
             ).reshape(bm, k, cin)
    att_b = (jnp.dot(xj2, axb_ref[...], preferred_element_type=jnp.float32)
             + jnp.dot(enc2, aeb_ref[...], preferred_element_type=jnp.float32)
             ).reshape(bm, k, ch)
    ea = jnp.exp(att_a - jnp.max(att_a, axis=1, keepdims=True))
    sa = ea / jnp.sum(ea, axis=1, keepdims=True)
    eb = jnp.exp(att_b - jnp.max(att_b, axis=1, keepdims=True))
    sb = eb / jnp.sum(eb, axis=1, keepdims=True)
    agg_x = jnp.sum(sa * xj, axis=1)
    agg_e = jnp.sum(sb * enc, axis=1)
    out = (jnp.dot(agg_x, pwx_ref[...], preferred_element_type=jnp.float32)
           + jnp.dot(agg_e, pwe_ref[...], preferred_element_type=jnp.float32)
           + pb_ref[...])
    o_ref[...] = jnp.maximum(out, 0.0)


def _lfa(pp, x, pos, flat_nbr):
    n, cin = x.shape
    c = 2 * cin
    table = jnp.concatenate([x, pos], axis=1)
    g = _sc_gather(table, flat_nbr)
    xj = g[:, :cin].reshape(n, K_NBR, cin)
    pj = g[:, cin:cin + 3].reshape(n, K_NBR, 3)

    ew, ebias = pp["enc"]["W"], pp["enc"]["b"].reshape(1, -1)
    aw = pp["att"]["W"]
    pw, pbias = pp["post"]["W"], pp["post"]["b"].reshape(1, -1)
    bm = min(n, 256)
    grid = n // bm
    row = lambda i: (i, 0)
    row3 = lambda i: (i, 0, 0)
    full = lambda i: (0, 0)
    return pl.pallas_call(
        functools.partial(_lfa_body, bm, cin),
        grid=(grid,),
        in_specs=[
            pl.BlockSpec((bm, cin), row),
            pl.BlockSpec((bm, K_NBR, cin), row3),
            pl.BlockSpec((bm, 3), row),
            pl.BlockSpec((bm, K_NBR, 3), row3),
            pl.BlockSpec((3, cin), full),
            pl.BlockSpec((3, cin), full),
            pl.BlockSpec((3, cin), full),
            pl.BlockSpec((1, cin), full),
            pl.BlockSpec((1, cin), full),
            pl.BlockSpec((cin, cin), full),
            pl.BlockSpec((cin, cin), full),
            pl.BlockSpec((cin, cin), full),
            pl.BlockSpec((cin, cin), full),
            pl.BlockSpec((cin, c), full),
            pl.BlockSpec((cin, c), full),
            pl.BlockSpec((1, c), full),
        ],
        out_specs=pl.BlockSpec((bm, c), row),
        out_shape=jax.ShapeDtypeStruct((n, c), jnp.float32),
    )(x, xj, pos, pj,
      ew[0:3], ew[3:6], ew[6:9], ew[9:10], ebias,
      aw[:cin, :cin], aw[:cin, cin:], aw[cin:, :cin], aw[cin:, cin:],
      pw[:cin], pw[cin:], pbias)


def _linear_body(act, x_ref, w_ref, b_ref, o_ref):
    y = jnp.dot(x_ref[...], w_ref[...],
                preferred_element_type=jnp.float32) + b_ref[...]
    if act == "relu":
        y = jnp.maximum(y, 0.0)
    elif act == "logsoftmax":
        y = y - jnp.max(y, axis=1, keepdims=True)
        y = y - jnp.log(jnp.sum(jnp.exp(y), axis=1, keepdims=True))
    o_ref[...] = y


def _linear(pp, x, act="none"):
    n = x.shape[0]
    do = pp["W"].shape[1]
    return pl.pallas_call(
        functools.partial(_linear_body, act),
        out_shape=jax.ShapeDtypeStruct((n, do), jnp.float32),
    )(x, pp["W"], pp["b"].reshape(1, do))


def _drb_tail_body(h_ref, x_ref, w2_ref, b2_ref, ws_ref, bs_ref, o_ref):
    y = (jnp.dot(h_ref[...], w2_ref[...], preferred_element_type=jnp.float32)
         + b2_ref[...]
         + jnp.dot(x_ref[...], ws_ref[...], preferred_element_type=jnp.float32)
         + bs_ref[...])
    o_ref[...] = jnp.where(y > 0, y, 0.01 * y)


def _drb_tail(pp2, pps, h, x):
    n = h.shape[0]
    do = pp2["W"].shape[1]
    return pl.pallas_call(
        _drb_tail_body,
        out_shape=jax.ShapeDtypeStruct((n, do), jnp.float32),
    )(h, x, pp2["W"], pp2["b"].reshape(1, do),
      pps["W"], pps["b"].reshape(1, do))


def _fp2_body(ga_ref, gb_ref, xs_ref, w1a_ref, w2a_ref, ba_ref,
              w1b_ref, w2b_ref, bb_ref, oa_ref, ob_ref):
    xs = xs_ref[...]
    ya = (jnp.dot(ga_ref[...], w1a_ref[...], preferred_element_type=jnp.float32)
          + jnp.dot(xs, w2a_ref[...], preferred_element_type=jnp.float32)
          + ba_ref[...])
    yb = (jnp.dot(gb_ref[...], w1b_ref[...], preferred_element_type=jnp.float32)
          + jnp.dot(xs, w2b_ref[...], preferred_element_type=jnp.float32)
          + bb_ref[...])
    oa_ref[...] = jnp.maximum(ya, 0.0)
    ob_ref[...] = jnp.maximum(yb, 0.0)


def _fp2(ppa, ppb, ga, gb, x_skip):
    n, din = ga.shape
    do = ppa["W"].shape[1]
    return pl.pallas_call(
        _fp2_body,
        out_shape=(jax.ShapeDtypeStruct((n, do), jnp.float32),
                   jax.ShapeDtypeStruct((n, do), jnp.float32)),
    )(ga, gb, x_skip,
      ppa["W"][:din], ppa["W"][din:], ppa["b"].reshape(1, do),
      ppb["W"][:din], ppb["W"][din:], ppb["b"].reshape(1, do))


def _heads_body(sem_ref, inst_ref, w1_ref, b1_ref, w2_ref, b2_ref,
                wc_ref, bc_ref, wi_ref, bi_ref, os_ref, oi_ref):
    h = jnp.maximum(
        jnp.dot(sem_ref[...], w1_ref[...],
                preferred_element_type=jnp.float32) + b1_ref[...], 0.0)
    h = jnp.maximum(
        jnp.dot(h, w2_ref[...],
                preferred_element_type=jnp.float32) + b2_ref[...], 0.0)
    y = jnp.dot(h, wc_ref[...], preferred_element_type=jnp.float32) + bc_ref[...]
    y = y - jnp.max(y, axis=1, keepdims=True)
    os_ref[...] = y - jnp.log(jnp.sum(jnp.exp(y), axis=1, keepdims=True))
    oi_ref[...] = (jnp.dot(inst_ref[...], wi_ref[...],
                           preferred_element_type=jnp.float32) + bi_ref[...])


def _heads(params, sem, inst):
    n = sem.shape[0]
    p1, p2 = params["clf1"], params["clf2"]
    pc, pi = params["fc_classif"], params["fc_inst"]
    return pl.pallas_call(
        _heads_body,
        out_shape=(jax.ShapeDtypeStruct((n, pc["W"].shape[1]), jnp.float32),
                   jax.ShapeDtypeStruct((n, pi["W"].shape[1]), jnp.float32)),
    )(sem, inst,
      p1["W"], p1["b"].reshape(1, -1), p2["W"], p2["b"].reshape(1, -1),
      pc["W"], pc["b"].reshape(1, -1), pi["W"], pi["b"].reshape(1, -1))


def _drb(pp, x, pos):
    nbr = _knn(pos, pos, K_NBR)
    flat = nbr.reshape(-1)
    h = _linear(pp["mlp1"], x, act="relu")
    h = _lfa(pp["lfa1"], h, pos, flat)
    h = _lfa(pp["lfa2"], h, pos, flat)
    return _drb_tail(pp["mlp2"], pp["short"], h, x)


def _decim(n, salt):
    perm = jax.random.permutation(
        jax.random.fold_in(jax.random.key(7), salt), n)
    return perm[: n // 4]


def kernel(x, pos, batch, ptr, params):
    n = x.shape[0]
    h = _linear(params["fc0"], x)
    h1 = _drb(params["b1"], h, pos)
    i1 = _decim(n, 1)
    h1d = h1[i1]
    p1d = pos[i1]
    h2 = _drb(params["b2"], h1d, p1d)
    i2 = _decim(n // 4, 2)
    h2d = h2[i2]
    p2d = p1d[i2]
    h3 = _drb(params["b3"], h2d, p2d)
    i3 = _decim(n // 16, 3)
    h3d = h3[i3]
    p3d = p2d[i3]
    h4 = _drb(params["b4"], h3d, p3d)
    i4 = _decim(n // 64, 4)
    h4d = h4[i4]
    p4d = p3d[i4]
    seed_idx = jnp.arange(n)[i1][i2][i3][i4]
    s = _linear(params["summit"], h4d, act="relu")

    ni4 = _knn(p3d, p4d, 1)[:, 0]
    ni3 = _knn(p2d, p3d, 1)[:, 0]
    ni2 = _knn(p1d, p2d, 1)[:, 0]
    ni1 = _knn(pos, p1d, 1)[:, 0]

    g4 = _sc_gather(s, ni4)
    sem, inst = _fp2(params["sem_fp4"], params["inst_fp4"], g4, g4, h3d)
    for lvl, ni, skip in (("3", ni3, h2d), ("2", ni2, h1d), ("1", ni1, h1)):
        c = sem.shape[1]
        g = _sc_gather(jnp.concatenate([sem, inst], axis=1), ni)
        sem, inst = _fp2(params["sem_fp" + lvl], params["inst_fp" + lvl],
                         g[:, :c], g[:, c:], skip)
    return (*_heads(params, sem, inst), seed_idx)


# speedup vs baseline: 4.9333x; 1.1914x over previous
import functools

import jax
import jax.numpy as jnp
from jax import lax
from jax.experimental import pallas as pl
from jax.experimental.pallas import tpu as pltpu
from jax.experimental.pallas import tpu_sc as plsc

K_NBR = 16

_SC_NC = 2
_SC_NS = 16
_SC_NW = _SC_NC * _SC_NS


def _sc_gather(table, idx):
    (b0,) = idx.shape
    b_total = (b0 + 8 * _SC_NW - 1) // (8 * _SC_NW) * (8 * _SC_NW)
    if b_total != b0:
        idx = jnp.concatenate(
            [idx, jnp.zeros((b_total - b0,), idx.dtype)])
    n, d0 = table.shape
    d = (d0 + 127) // 128 * 128
    if d != d0:
        table = jnp.concatenate(
            [table, jnp.zeros((n, d - d0), jnp.float32)], axis=1)
    bpw = b_total // _SC_NW
    cs = min(bpw, max(8, 262144 // (d * 4)))
    while bpw % cs:
        cs -= 1
    iters = bpw // cs
    mesh = plsc.VectorSubcoreMesh(core_axis_name="c", subcore_axis_name="s")

    @functools.partial(
        pl.kernel,
        mesh=mesh,
        out_type=jax.ShapeDtypeStruct((b_total, d), jnp.float32),
        scratch_types=[
            pltpu.VMEM((cs,), jnp.int32),
            pltpu.VMEM((cs, d), jnp.float32),
            pltpu.SemaphoreType.DMA,
        ],
    )
    def k(table_hbm, idx_hbm, out_hbm, idx_v, rows_v, sem):
        wid = lax.axis_index("s") * _SC_NC + lax.axis_index("c")
        base = wid * bpw

        def body(t, carry):
            off = base + t * cs
            pltpu.sync_copy(idx_hbm.at[pl.ds(off, cs)], idx_v)
            pltpu.async_copy(table_hbm.at[idx_v], rows_v, sem).wait()
            pltpu.sync_copy(rows_v, out_hbm.at[pl.ds(off, cs)])
            return carry

        lax.fori_loop(0, iters, body, 0)

    out = k(table, idx)
    return out[:b0, :d0]


def _extract_min_cols(d, idx_of, k):
    big = jnp.int32(2**30)
    vcols, icols = [], []
    for j in range(k):
        m = jnp.min(d, axis=1, keepdims=True)
        sel = jnp.min(jnp.where(d <= m, idx_of, big), axis=1, keepdims=True)
        vcols.append(m)
        icols.append(sel)
        if j + 1 < k:
            d = jnp.where(idx_of == sel, jnp.float32(jnp.inf), d)
    if k == 1:
        return vcols[0], icols[0]
    return jnp.concatenate(vcols, axis=1), jnp.concatenate(icols, axis=1)


def _knn_body(k, chunk, q_ref, r_ref, o_ref):
    nr = r_ref.shape[0]
    q = q_ref[...]
    qsq = jnp.sum(q * q, axis=1, keepdims=True)
    cand_v, cand_i = [], []
    for c in range(nr // chunk):
        r = r_ref[pl.ds(c * chunk, chunk), :]
        rsq = jnp.sum(r * r, axis=1)[None, :]
        d = qsq - 2.0 * jnp.dot(q, r.T,
                                preferred_element_type=jnp.float32) + rsq
        iota = lax.broadcasted_iota(jnp.int32, d.shape, 1) + c * chunk
        kk = min(k, chunk)
        v, i = _extract_min_cols(d, iota, kk)
        cand_v.append(v)
        cand_i.append(i)
    if len(cand_v) == 1:
        o_ref[...] = cand_i[0][:, :k]
        return
    vs = jnp.concatenate(cand_v, axis=1)
    xs = jnp.concatenate(cand_i, axis=1)
    _, sel = _extract_min_cols(vs, xs, k)
    o_ref[...] = sel


def _knn(query, ref, k):
    nq = query.shape[0]
    nr = ref.shape[0]
    bm = min(nq, max(256, 2**22 // (4 * nr)))
    grid = nq // bm
    chunk = 512 if (nr % 512 == 0 and nr >= 512) else nr
    return pl.pallas_call(
        functools.partial(_knn_body, k, chunk),
        grid=(grid,),
        in_specs=[
            pl.BlockSpec((bm, 3), lambda i: (i, 0)),
            pl.BlockSpec((nr, 3), lambda i: (0, 0)),
        ],
        out_specs=pl.BlockSpec((bm, k), lambda i: (i, 0)),
        out_shape=jax.ShapeDtypeStruct((nq, k), jnp.int32),
    )(query, ref)


def _lfa_body(bm, cin, x_ref, xj_ref, pi_ref, pj_ref,
              wpi_ref, wpj_ref, wd_ref, weu_ref, eb_ref,
              axa_ref, axb_ref, aea_ref, aeb_ref,
              pwx_ref, pwe_ref, pb_ref, o_ref):
    k = K_NBR
    ch = cin
    c = 2 * cin
    pi = pi_ref[...]
    pj = pj_ref[...]
    dv = pj - pi[:, None, :]
    eu = jnp.sqrt(jnp.sum(dv * dv, axis=2, keepdims=True) + 1e-12)
    pj2 = pj.reshape(bm * k, 3)
    dv2 = dv.reshape(bm * k, 3)
    eu2 = eu.reshape(bm * k, 1)
    enc2 = (jnp.dot(pj2, wpj_ref[...], preferred_element_type=jnp.float32)
            + jnp.dot(dv2, wd_ref[...], preferred_element_type=jnp.float32)
            + eu2 * weu_ref[...]
            + eb_ref[...])
    enc_pi = jnp.dot(pi, wpi_ref[...], preferred_element_type=jnp.float32)
    enc = jnp.maximum(enc2.reshape(bm, k, ch) + enc_pi[:, None, :], 0.0)
    enc2 = enc.reshape(bm * k, ch)
    xj = xj_ref[...]
    xj2 = xj.reshape(bm * k, cin)

    att_a = (jnp.dot(xj2, axa_ref[...], preferred_element_type=jnp.float32)
             + jnp.dot(enc2, aea_ref[...], preferred_element_type=jnp.float32)
             ).reshape(bm, k, cin)
    att_b = (jnp.dot(xj2, axb_ref[...], preferred_element_type=jnp.float32)
             + jnp.dot(enc2, aeb_ref[...], preferred_element_type=jnp.float32)
             ).reshape(bm, k, ch)
    ea = jnp.exp(att_a - jnp.max(att_a, axis=1, keepdims=True))
    sa = ea / jnp.sum(ea, axis=1, keepdims=True)
    eb = jnp.exp(att_b - jnp.max(att_b, axis=1, keepdims=True))
    sb = eb / jnp.sum(eb, axis=1, keepdims=True)
    agg_x = jnp.sum(sa * xj, axis=1)
    agg_e = jnp.sum(sb * enc, axis=1)
    out = (jnp.dot(agg_x, pwx_ref[...], preferred_element_type=jnp.float32)
           + jnp.dot(agg_e, pwe_ref[...], preferred_element_type=jnp.float32)
           + pb_ref[...])
    o_ref[...] = jnp.maximum(out, 0.0)


def _lfa(pp, x, pos, flat_nbr):
    n, cin = x.shape
    c = 2 * cin
    table = jnp.concatenate([x, pos], axis=1)
    g = _sc_gather(table, flat_nbr)
    xj = g[:, :cin].reshape(n, K_NBR, cin)
    pj = g[:, cin:cin + 3].reshape(n, K_NBR, 3)

    ew, ebias = pp["enc"]["W"], pp["enc"]["b"].reshape(1, -1)
    aw = pp["att"]["W"]
    pw, pbias = pp["post"]["W"], pp["post"]["b"].reshape(1, -1)
    bm = min(n, 256)
    grid = n // bm
    row = lambda i: (i, 0)
    row3 = lambda i: (i, 0, 0)
    full = lambda i: (0, 0)
    return pl.pallas_call(
        functools.partial(_lfa_body, bm, cin),
        grid=(grid,),
        in_specs=[
            pl.BlockSpec((bm, cin), row),
            pl.BlockSpec((bm, K_NBR, cin), row3),
            pl.BlockSpec((bm, 3), row),
            pl.BlockSpec((bm, K_NBR, 3), row3),
            pl.BlockSpec((3, cin), full),
            pl.BlockSpec((3, cin), full),
            pl.BlockSpec((3, cin), full),
            pl.BlockSpec((1, cin), full),
            pl.BlockSpec((1, cin), full),
            pl.BlockSpec((cin, cin), full),
            pl.BlockSpec((cin, cin), full),
            pl.BlockSpec((cin, cin), full),
            pl.BlockSpec((cin, cin), full),
            pl.BlockSpec((cin, c), full),
            pl.BlockSpec((cin, c), full),
            pl.BlockSpec((1, c), full),
        ],
        out_specs=pl.BlockSpec((bm, c), row),
        out_shape=jax.ShapeDtypeStruct((n, c), jnp.float32),
    )(x, xj, pos, pj,
      ew[0:3], ew[3:6], ew[6:9], ew[9:10], ebias,
      aw[:cin, :cin], aw[:cin, cin:], aw[cin:, :cin], aw[cin:, cin:],
      pw[:cin], pw[cin:], pbias)


def _linear_body(act, x_ref, w_ref, b_ref, o_ref):
    y = jnp.dot(x_ref[...], w_ref[...],
                preferred_element_type=jnp.float32) + b_ref[...]
    if act == "relu":
        y = jnp.maximum(y, 0.0)
    elif act == "logsoftmax":
        y = y - jnp.max(y, axis=1, keepdims=True)
        y = y - jnp.log(jnp.sum(jnp.exp(y), axis=1, keepdims=True))
    o_ref[...] = y


def _linear(pp, x, act="none"):
    n = x.shape[0]
    do = pp["W"].shape[1]
    return pl.pallas_call(
        functools.partial(_linear_body, act),
        out_shape=jax.ShapeDtypeStruct((n, do), jnp.float32),
    )(x, pp["W"], pp["b"].reshape(1, do))


def _drb_tail_body(h_ref, x_ref, w2_ref, b2_ref, ws_ref, bs_ref, o_ref):
    y = (jnp.dot(h_ref[...], w2_ref[...], preferred_element_type=jnp.float32)
         + b2_ref[...]
         + jnp.dot(x_ref[...], ws_ref[...], preferred_element_type=jnp.float32)
         + bs_ref[...])
    o_ref[...] = jnp.where(y > 0, y, 0.01 * y)


def _drb_tail(pp2, pps, h, x):
    n = h.shape[0]
    do = pp2["W"].shape[1]
    return pl.pallas_call(
        _drb_tail_body,
        out_shape=jax.ShapeDtypeStruct((n, do), jnp.float32),
    )(h, x, pp2["W"], pp2["b"].reshape(1, do),
      pps["W"], pps["b"].reshape(1, do))


def _fp2_body(ga_ref, gb_ref, xs_ref, w1a_ref, w2a_ref, ba_ref,
              w1b_ref, w2b_ref, bb_ref, oa_ref, ob_ref):
    xs = xs_ref[...]
    ya = (jnp.dot(ga_ref[...], w1a_ref[...], preferred_element_type=jnp.float32)
          + jnp.dot(xs, w2a_ref[...], preferred_element_type=jnp.float32)
          + ba_ref[...])
    yb = (jnp.dot(gb_ref[...], w1b_ref[...], preferred_element_type=jnp.float32)
          + jnp.dot(xs, w2b_ref[...], preferred_element_type=jnp.float32)
          + bb_ref[...])
    oa_ref[...] = jnp.maximum(ya, 0.0)
    ob_ref[...] = jnp.maximum(yb, 0.0)


def _fp2(ppa, ppb, ga, gb, x_skip):
    n, din = ga.shape
    do = ppa["W"].shape[1]
    return pl.pallas_call(
        _fp2_body,
        out_shape=(jax.ShapeDtypeStruct((n, do), jnp.float32),
                   jax.ShapeDtypeStruct((n, do), jnp.float32)),
    )(ga, gb, x_skip,
      ppa["W"][:din], ppa["W"][din:], ppa["b"].reshape(1, do),
      ppb["W"][:din], ppb["W"][din:], ppb["b"].reshape(1, do))


def _heads_body(sem_ref, inst_ref, w1_ref, b1_ref, w2_ref, b2_ref,
                wc_ref, bc_ref, wi_ref, bi_ref, os_ref, oi_ref):
    h = jnp.maximum(
        jnp.dot(sem_ref[...], w1_ref[...],
                preferred_element_type=jnp.float32) + b1_ref[...], 0.0)
    h = jnp.maximum(
        jnp.dot(h, w2_ref[...],
                preferred_element_type=jnp.float32) + b2_ref[...], 0.0)
    y = jnp.dot(h, wc_ref[...], preferred_element_type=jnp.float32) + bc_ref[...]
    y = y - jnp.max(y, axis=1, keepdims=True)
    os_ref[...] = y - jnp.log(jnp.sum(jnp.exp(y), axis=1, keepdims=True))
    oi_ref[...] = (jnp.dot(inst_ref[...], wi_ref[...],
                           preferred_element_type=jnp.float32) + bi_ref[...])


def _heads(params, sem, inst):
    n = sem.shape[0]
    p1, p2 = params["clf1"], params["clf2"]
    pc, pi = params["fc_classif"], params["fc_inst"]
    return pl.pallas_call(
        _heads_body,
        out_shape=(jax.ShapeDtypeStruct((n, pc["W"].shape[1]), jnp.float32),
                   jax.ShapeDtypeStruct((n, pi["W"].shape[1]), jnp.float32)),
    )(sem, inst,
      p1["W"], p1["b"].reshape(1, -1), p2["W"], p2["b"].reshape(1, -1),
      pc["W"], pc["b"].reshape(1, -1), pi["W"], pi["b"].reshape(1, -1))


def _drb(pp, x, pos):
    nbr = _knn(pos, pos, K_NBR)
    flat = nbr.reshape(-1)
    h = _linear(pp["mlp1"], x, act="relu")
    h = _lfa(pp["lfa1"], h, pos, flat)
    h = _lfa(pp["lfa2"], h, pos, flat)
    return _drb_tail(pp["mlp2"], pp["short"], h, x)


def _decim(n, salt):
    perm = jax.random.permutation(
        jax.random.fold_in(jax.random.key(7), salt), n)
    return perm[: n // 4]


def kernel(x, pos, batch, ptr, params):
    n = x.shape[0]
    h = _linear(params["fc0"], x)
    h1 = _drb(params["b1"], h, pos)
    i1 = _decim(n, 1)
    h1d = h1[i1]
    p1d = pos[i1]
    h2 = _drb(params["b2"], h1d, p1d)
    i2 = _decim(n // 4, 2)
    h2d = h2[i2]
    p2d = p1d[i2]
    h3 = _drb(params["b3"], h2d, p2d)
    i3 = _decim(n // 16, 3)
    h3d = h3[i3]
    p3d = p2d[i3]
    h4 = _drb(params["b4"], h3d, p3d)
    i4 = _decim(n // 64, 4)
    h4d = h4[i4]
    p4d = p3d[i4]
    seed_idx = jnp.arange(n)[i1][i2][i3][i4]
    s = _linear(params["summit"], h4d, act="relu")

    ni4 = _knn(p3d, p4d, 1)[:, 0]
    ni3 = _knn(p2d, p3d, 1)[:, 0]
    ni2 = _knn(p1d, p2d, 1)[:, 0]
    ni1 = _knn(pos, p1d, 1)[:, 0]

    g4 = _sc_gather(s, ni4)
    sem, inst = _fp2(params["sem_fp4"], params["inst_fp4"], g4, g4, h3d)
    for lvl, ni, skip in (("3", ni3, h2d), ("2", ni2, h1d), ("1", ni1, h1)):
        c = sem.shape[1]
        g = _sc_gather(jnp.concatenate([sem, inst], axis=1), ni)
        sem, inst = _fp2(params["sem_fp" + lvl], params["inst_fp" + lvl],
                         g[:, :c], g[:, c:], skip)
    return (*_heads(params, sem, inst), seed_idx)
